# baseline (speedup 1.0000x reference)
import functools

import jax
import jax.numpy as jnp
import numpy as np
from jax import lax
from jax.experimental import pallas as pl
from jax.experimental.pallas import tpu as pltpu
from jax.experimental.pallas import tpu_sc as plsc

N = 10000
E = 320000
D = 128
H = 8
DK = 16
R = 100
DFF = 4 * D
NC, NS = 2, 16
NW = NC * NS
CH = 32
NCHUNK = 314
EPT = NCHUNK * CH
IW = 128
NPAD = 10112
VPT = NPAD // NS

BN = 1000
_DN = (((1,), (1,)), ((), ()))


def _ln(t, g, b):
    m = jnp.mean(t, axis=-1, keepdims=True)
    v = jnp.mean((t - m) ** 2, axis=-1, keepdims=True)
    return (t - m) / jnp.sqrt(v + 1e-5) * g + b


def _front_body(x_ref, wq_ref, bq_ref, wk_ref, wv_ref, q_ref, k_ref, v_ref):
    x = x_ref[...]
    f32 = jnp.float32
    q_ref[...] = lax.dot_general(x, wq_ref[...], _DN, preferred_element_type=f32) + bq_ref[...]
    k_ref[...] = lax.dot_general(x, wk_ref[...], _DN, preferred_element_type=f32)
    v_ref[...] = lax.dot_general(x, wv_ref[...], _DN, preferred_element_type=f32)


def _tc_front(x, Wq, bq, Wk, Wv):
    rows = pl.BlockSpec((BN, D), lambda i: (i, 0))
    wsp = pl.BlockSpec((D, D), lambda i: (0, 0))
    bsp = pl.BlockSpec((1, D), lambda i: (0, 0))
    return pl.pallas_call(
        _front_body,
        grid=(N // BN,),
        in_specs=[rows, wsp, bsp, wsp, wsp],
        out_specs=[rows, rows, rows],
        out_shape=[jax.ShapeDtypeStruct((N, D), jnp.float32)] * 3,
    )(x, Wq, bq.reshape(1, D), Wk, Wv)


def _post_body(with_qkv, x_ref, pv_ref, pz_ref, wo_ref, bo_ref, g1_ref, be1_ref,
               w1_ref, b1_ref, w2_ref, b2_ref, g2_ref, be2_ref, *rest):
    if with_qkv:
        wqn_ref, bqn_ref, wkn_ref, wvn_ref, xo_ref, qo_ref, ko_ref, vo_ref = rest
    else:
        (xo_ref,) = rest
    f32 = jnp.float32
    wv = pv_ref[0] + pv_ref[1]
    z = (pz_ref[0] + pz_ref[1])[:, :H]
    hrow = lax.broadcasted_iota(jnp.int32, (H, D), 0)
    hcol = lax.broadcasted_iota(jnp.int32, (H, D), 1) // DK
    bmat = (hrow == hcol).astype(f32)
    zexp = lax.dot_general(z, bmat, (((1,), (0,)), ((), ())), preferred_element_type=f32)
    o = wv / zexp
    t = x_ref[...] + lax.dot_general(o, wo_ref[...], _DN, preferred_element_type=f32) + bo_ref[...]
    h = _ln(t, g1_ref[...], be1_ref[...])
    ff = jnp.maximum(lax.dot_general(h, w1_ref[...], _DN, preferred_element_type=f32) + b1_ref[...], 0.0)
    ff2 = lax.dot_general(ff, w2_ref[...], _DN, preferred_element_type=f32) + b2_ref[...]
    xn = _ln(h + ff2, g2_ref[...], be2_ref[...])
    xo_ref[...] = xn
    if with_qkv:
        qo_ref[...] = lax.dot_general(xn, wqn_ref[...], _DN, preferred_element_type=f32) + bqn_ref[...]
        ko_ref[...] = lax.dot_general(xn, wkn_ref[...], _DN, preferred_element_type=f32)
        vo_ref[...] = lax.dot_general(xn, wvn_ref[...], _DN, preferred_element_type=f32)


def _tc_post(x, accv, accz, Wo, bo, g1, be1, W1, b1, W2, b2, g2, be2, nxt=None):
    rows = pl.BlockSpec((BN, D), lambda i: (i, 0))
    pvsp = pl.BlockSpec((NC, BN, D), lambda i: (0, i, 0))
    pzsp = pl.BlockSpec((NC, BN, DK), lambda i: (0, i, 0))
    wsp = pl.BlockSpec((D, D), lambda i: (0, 0))
    bsp = pl.BlockSpec((1, D), lambda i: (0, 0))
    w1sp = pl.BlockSpec((DFF, D), lambda i: (0, 0))
    b1sp = pl.BlockSpec((1, DFF), lambda i: (0, 0))
    w2sp = pl.BlockSpec((D, DFF), lambda i: (0, 0))
    in_specs = [rows, pvsp, pzsp, wsp, bsp, bsp, bsp, w1sp, b1sp, w2sp, bsp, bsp, bsp]
    args = [x, accv, accz, Wo, bo.reshape(1, D), g1.reshape(1, D), be1.reshape(1, D),
            W1, b1.reshape(1, DFF), W2, b2.reshape(1, D), g2.reshape(1, D), be2.reshape(1, D)]
    with_qkv = nxt is not None
    nout = 4 if with_qkv else 1
    if with_qkv:
        Wqn, bqn, Wkn, Wvn = nxt
        in_specs += [wsp, bsp, wsp, wsp]
        args += [Wqn, bqn.reshape(1, D), Wkn, Wvn]
    return pl.pallas_call(
        functools.partial(_post_body, with_qkv),
        grid=(N // BN,),
        in_specs=in_specs,
        out_specs=[rows] * nout,
        out_shape=[jax.ShapeDtypeStruct((N, D), jnp.float32)] * nout,
    )(*args)


def _make_perms(lane):
    def c2(v):
        return v.reshape(16, 1)
    return {
        "rot8": c2((lane + 8) & 15),
        "w8r4": c2((lane & 8) | ((lane + 4) & 7)),
        "l2": c2((lane & 3) | ((lane & 4) << 1)),
        "w4r2": c2((lane & 12) | ((lane + 2) & 3)),
        "l3": c2((((lane & 7) >> 1) << 2) | (lane & 1)),
        "w2r1": c2(lane ^ 1),
        "cmp": c2((lane & 7) << 1),
    }


_GDN = lax.GatherDimensionNumbers(
    offset_dims=(), collapsed_slice_dims=(0,), start_index_map=(0,))


def _perm(x, idx):
    return lax.gather(x, idx, _GDN, (1,),
                      mode=lax.GatherScatterMode.PROMISE_IN_BOUNDS)


def _sum8(ts, m8, P):
    a = []
    for i in range(4):
        lo = jnp.where(m8, ts[2 * i], _perm(ts[2 * i + 1], P["rot8"]))
        hi = jnp.where(m8, _perm(ts[2 * i], P["rot8"]), ts[2 * i + 1])
        a.append(lo + hi)
    c = []
    for j in range(2):
        f0 = a[2 * j] + _perm(a[2 * j], P["w8r4"])
        f1 = a[2 * j + 1] + _perm(a[2 * j + 1], P["w8r4"])
        c.append(jnp.where(m8, _perm(f0, P["l2"]), _perm(f1, P["l2"])))
    f0 = c[0] + _perm(c[0], P["w4r2"])
    f1 = c[1] + _perm(c[1], P["w4r2"])
    d = jnp.where(m8, _perm(f0, P["l3"]), _perm(f1, P["l3"]))
    s = d + _perm(d, P["w2r1"])
    return _perm(s, P["cmp"])


def _edge_body(q_hbm, k_hbm, v_hbm, idx_hbm, re_hbm,
               accv_hbm, accz_hbm,
               accv_sh, accz_sh,
               idx_a, k_a, q_a, v_v,
               dst_v, re_v, out_z):
    c = lax.axis_index("c")
    s = lax.axis_index("s")
    wid = c * NS + s
    f32 = jnp.float32
    zero16 = jnp.zeros((16,), f32)

    def _zrow(r, carry):
        for j in range(D // 16):
            k_a[r, pl.ds(j * 16, 16)] = zero16
        out_z[r, pl.ds(0, 16)] = zero16
        return carry
    lax.fori_loop(0, CH, _zrow, 0)

    def _zstage(t, carry):
        pltpu.sync_copy(k_a, accv_sh.at[pl.ds(s * VPT + t * CH, CH)])
        pltpu.sync_copy(out_z, accz_sh.at[pl.ds(s * VPT + t * CH, CH)])
        return carry
    lax.fori_loop(0, VPT // CH, _zstage, 0)
    _rem = VPT - (VPT // CH) * CH
    if _rem:
        _ro = s * VPT + (VPT // CH) * CH
        pltpu.sync_copy(k_a.at[pl.ds(0, _rem)], accv_sh.at[pl.ds(_ro, _rem)])
        pltpu.sync_copy(out_z.at[pl.ds(0, _rem)], accz_sh.at[pl.ds(_ro, _rem)])
    plsc.subcore_barrier()

    lane = lax.iota(jnp.int32, 16)
    m8 = lane < H
    P = _make_perms(lane)
    hb_idx = [jnp.full((16, 1), h, jnp.int32) for h in range(H)]
    pltpu.sync_copy(re_hbm, re_v)

    base0 = wid * NCHUNK

    def _load_idx(ci, idx_v):
        pltpu.sync_copy(idx_hbm.at[pl.ds((base0 + ci) * IW, IW)], idx_v)

    def _score(idx_v, k_v, q_v):
        def _group(g, gcarry):
            rel_row = idx_v[pl.ds(2 * CH + g * 8, 16)]
            for j in range(8):
                e = g * 8 + j
                rid = rel_row[j]
                lgx = re_v[rid, :]
                ts = []
                for h in range(H):
                    kc = k_v[e, pl.ds(h * DK, DK)]
                    qc = q_v[e, pl.ds(h * DK, DK)]
                    ts.append((kc + lgx) * qc)
                svec = _sum8(ts, m8, P)
                svec = jnp.exp(jnp.clip(svec * 0.25, -10.0, 10.0))
                out_z[e, pl.ds(0, 16)] = jnp.where(m8, svec, 0.0)
            return gcarry
        lax.fori_loop(0, CH // 8, _group, 0)

    def _wv_scatter(idx_v, k_v):
        def _group(g, gcarry):
            rel_row = idx_v[pl.ds(2 * CH + g * 8, 16)]
            for j in range(8):
                e = g * 8 + j
                rid = rel_row[j]
                lgx = re_v[rid, :]
                svec = out_z[e, pl.ds(0, 16)]
                for h in range(H):
                    wb = _perm(svec, hb_idx[h])
                    vc = v_v[e, pl.ds(h * DK, DK)]
                    k_v[e, pl.ds(h * DK, DK)] = (vc + lgx) * wb
            return gcarry
        lax.fori_loop(0, CH // 8, _group, 0)
        dst_v[pl.ds(0, 16)] = idx_v[pl.ds(CH, 16)]
        dst_v[pl.ds(16, 16)] = idx_v[pl.ds(CH + 16, 16)]
        pltpu.sync_copy(k_v, accv_sh.at[dst_v], add=True)
        pltpu.sync_copy(out_z, accz_sh.at[dst_v], add=True)

    def _chunk(ci, carry):
        _load_idx(ci, idx_a)
        srcs = idx_a.at[pl.ds(0, CH)]
        dsts = idx_a.at[pl.ds(CH, CH)]
        pltpu.sync_copy(k_hbm.at[srcs], k_a)
        pltpu.sync_copy(q_hbm.at[dsts], q_a)
        pltpu.sync_copy(v_hbm.at[srcs], v_v)
        _score(idx_a, k_a, q_a)
        _wv_scatter(idx_a, k_a)
        return carry
    lax.fori_loop(0, NCHUNK, _chunk, 0)

    plsc.subcore_barrier()
    pltpu.sync_copy(accv_sh.at[pl.ds(s * VPT, VPT)],
                    accv_hbm.at[c, pl.ds(s * VPT, VPT)])
    def _zdump(t, carry):
        pltpu.sync_copy(accz_sh.at[pl.ds(s * VPT + t * CH, CH)], out_z)
        pltpu.sync_copy(out_z, accz_hbm.at[c, pl.ds(s * VPT + t * CH, CH)])
        return carry
    lax.fori_loop(0, VPT // CH, _zdump, 0)
    if _rem:
        pltpu.sync_copy(accz_sh.at[pl.ds(_ro, _rem)], out_z.at[pl.ds(0, _rem)])
        pltpu.sync_copy(out_z.at[pl.ds(0, _rem)], accz_hbm.at[c, pl.ds(_ro, _rem)])


def _sc_edge(q, k, v, idx_flat, rel_embed):
    f32 = jnp.float32
    mesh = plsc.VectorSubcoreMesh(core_axis_name="c", subcore_axis_name="s")
    fn = pl.kernel(
        _edge_body,
        out_type=[jax.ShapeDtypeStruct((NC, NPAD, D), f32),
                  jax.ShapeDtypeStruct((NC, NPAD, DK), f32)],
        mesh=mesh,
        scratch_types=[
            pltpu.VMEM_SHARED((NPAD, D), f32),
            pltpu.VMEM_SHARED((NPAD, DK), f32),
            pltpu.VMEM((IW,), jnp.int32),
            pltpu.VMEM((CH, D), f32),
            pltpu.VMEM((CH, D), f32),
            pltpu.VMEM((CH, D), f32),
            pltpu.VMEM((CH,), jnp.int32),
            pltpu.VMEM((R, DK), f32),
            pltpu.VMEM((CH, DK), f32),
        ],
    )
    return fn(q, k, v, idx_flat, rel_embed)


def _jax_edge(q, k, v, edge_index, edges, rel_embed):
    src = edge_index[0]
    dst = edge_index[1]
    rel = rel_embed[edges]
    ke = k[src].reshape(E, H, DK) + rel[:, None, :]
    qe = q[dst].reshape(E, H, DK)
    s = jnp.clip(jnp.sum(ke * qe, -1) * 0.25, -10.0, 10.0)
    w = jnp.exp(s)
    ve = v[src].reshape(E, H, DK) + rel[:, None, :]
    wv = jax.ops.segment_sum((ve * w[:, :, None]).reshape(E, D), dst, N)
    z = jax.ops.segment_sum(w, dst, N)
    accv = jnp.zeros((NC, NPAD, D), jnp.float32).at[0, :N].set(wv)
    accz = jnp.zeros((NC, NPAD, DK), jnp.float32).at[0, :N, :H].set(z)
    return accv, accz


def kernel(x, edge_index, edges, rel_embed, Wq, bq, Wk, Wv, Wo, bo,
           ln1_g, ln1_b, W1, b1, W2, b2, ln2_g, ln2_b):
    i32 = jnp.int32
    npad_e = NW * EPT - E
    src = jnp.concatenate([edge_index[0].astype(i32), jnp.zeros((npad_e,), i32)])
    dst = jnp.concatenate([edge_index[1].astype(i32), jnp.full((npad_e,), N, i32)])
    rel = jnp.concatenate([edges.astype(i32), jnp.zeros((npad_e,), i32)])
    packed = jnp.concatenate(
        [src.reshape(NW * NCHUNK, CH), dst.reshape(NW * NCHUNK, CH),
         rel.reshape(NW * NCHUNK, CH),
         jnp.zeros((NW * NCHUNK, IW - 3 * CH), i32)], axis=1)
    idx_flat = jnp.concatenate([packed.reshape(-1), jnp.zeros((IW,), i32)])

    q0, k0, v0 = _tc_front(x, Wq[0], bq[0], Wk[0], Wv[0])
    accv, accz = _jax_edge(q0, k0, v0, edge_index, edges, rel_embed)
    x1, q1, k1, v1 = _tc_post(x, accv, accz, Wo[0], bo[0],
                              ln1_g[0], ln1_b[0], W1[0], b1[0], W2[0], b2[0],
                              ln2_g[0], ln2_b[0],
                              nxt=(Wq[1], bq[1], Wk[1], Wv[1]))
    accv, accz = _jax_edge(q1, k1, v1, edge_index, edges, rel_embed)
    (x2,) = _tc_post(x1, accv, accz, Wo[1], bo[1],
                     ln1_g[1], ln1_b[1], W1[1], b1[1], W2[1], b2[1],
                     ln2_g[1], ln2_b[1])
    return x2
